# baseline (speedup 1.0000x reference)
import functools

import jax
import jax.numpy as jnp
from jax import lax
from jax.experimental import pallas as pl
from jax.experimental.pallas import tpu as pltpu
from jax.experimental.pallas import tpu_sc as plsc

EMB = 64
NC = 2
NS = 16
NW = NC * NS
CHUNK = 128
NBUF = 4

_mesh = plsc.VectorSubcoreMesh(core_axis_name="c", subcore_axis_name="s")


def _make_gather(n_chunks: int):
    cpw = n_chunks // NW

    @functools.partial(
        pl.kernel,
        mesh=_mesh,
        out_type=jax.ShapeDtypeStruct((n_chunks, CHUNK, EMB), jnp.float32),
        scratch_types=[
            pltpu.VMEM((cpw, CHUNK), jnp.int32),
            [pltpu.VMEM((CHUNK, EMB), jnp.float32) for _ in range(NBUF)],
            [pltpu.SemaphoreType.DMA for _ in range(NBUF)],
            [pltpu.SemaphoreType.DMA for _ in range(NBUF)],
        ],
        compiler_params=pltpu.CompilerParams(use_tc_tiling_on_sc=False),
    )
    def gather_kernel(idx_hbm, table_hbm, out_hbm, idx_v, bufs, gsems, osems):
        wid = lax.axis_index("s") * NC + lax.axis_index("c")
        cbase = wid * cpw

        pltpu.sync_copy(idx_hbm.at[pl.ds(cbase, cpw)], idx_v)

        for b in range(NBUF):
            pltpu.async_copy(table_hbm.at[idx_v.at[b]], bufs[b], gsems[b])

        def group(g, _):
            for b in range(NBUF):
                i = g * NBUF + b
                pltpu.make_async_copy(
                    table_hbm.at[idx_v.at[i]], bufs[b], gsems[b]
                ).wait()
                pltpu.async_copy(bufs[b], out_hbm.at[cbase + i], osems[b])
                pltpu.make_async_copy(
                    bufs[b], out_hbm.at[cbase + i], osems[b]
                ).wait()
                pltpu.async_copy(
                    table_hbm.at[idx_v.at[i + NBUF]], bufs[b], gsems[b]
                )
            return _

        lax.fori_loop(0, (cpw - NBUF) // NBUF, group, None)

        for b in range(NBUF):
            i = cpw - NBUF + b
            pltpu.make_async_copy(
                table_hbm.at[idx_v.at[i]], bufs[b], gsems[b]
            ).wait()
            pltpu.async_copy(bufs[b], out_hbm.at[cbase + i], osems[b])
        for b in range(NBUF):
            i = cpw - NBUF + b
            pltpu.make_async_copy(
                bufs[b], out_hbm.at[cbase + i], osems[b]
            ).wait()

    return gather_kernel


def kernel(input_ids, token_table):
    batch, seq = input_ids.shape
    total = batch * seq
    n_chunks = total // CHUNK
    idx = jnp.reshape(input_ids.astype(jnp.int32), (n_chunks, CHUNK))
    out = _make_gather(n_chunks)(idx, token_table)
    return jnp.reshape(out, (batch, seq, EMB))
